# baseline (speedup 1.0000x reference)
import functools

import jax
import jax.numpy as jnp
from jax.experimental import pallas as pl
from jax.experimental.pallas import tpu as pltpu

_N_CHUNKS = 8


def _dma_body(in_ref, o_ref, sems):
    rows = o_ref.shape[0]
    chunk = rows // _N_CHUNKS
    copies = []
    for i in range(_N_CHUNKS):
        c = pltpu.make_async_copy(
            in_ref.at[pl.ds(i * chunk, chunk), :],
            o_ref.at[pl.ds(i * chunk, chunk), :],
            sems.at[i],
        )
        c.start()
        copies.append(c)
    for c in copies:
        c.wait()


def kernel(x, position_embeddings):
    seq_len = x.shape[1]
    d_model = position_embeddings.shape[1]
    out = pl.pallas_call(
        _dma_body,
        in_specs=[pl.BlockSpec(memory_space=pl.ANY)],
        out_specs=pl.BlockSpec(memory_space=pl.ANY),
        out_shape=jax.ShapeDtypeStruct((seq_len, d_model), position_embeddings.dtype),
        scratch_shapes=[pltpu.SemaphoreType.DMA((_N_CHUNKS,))],
    )(position_embeddings)
    return out[None, :, :]


# speedup vs baseline: 48.7902x; 48.7902x over previous
import jax
import jax.numpy as jnp
from jax.experimental import pallas as pl
from jax.experimental.pallas import tpu as pltpu


def _copy_block(in_ref, o_ref):
    o_ref[...] = in_ref[...]


def kernel(x, position_embeddings):
    seq_len = x.shape[1]
    d_model = position_embeddings.shape[1]
    block = 1024
    out = pl.pallas_call(
        _copy_block,
        grid=(seq_len // block,),
        in_specs=[pl.BlockSpec((block, d_model), lambda i: (i, 0))],
        out_specs=pl.BlockSpec((block, d_model), lambda i: (i, 0)),
        out_shape=jax.ShapeDtypeStruct((seq_len, d_model), position_embeddings.dtype),
        compiler_params=pltpu.CompilerParams(
            dimension_semantics=("parallel",),
        ),
    )(position_embeddings)
    return out[None, :, :]
